# baseline (speedup 1.0000x reference)
import functools

import jax
import jax.numpy as jnp
from jax import lax
from jax.experimental import pallas as pl
from jax.experimental.pallas import tpu as pltpu
from jax.experimental.pallas import tpu_sc as plsc

_VOCAB = 1000000
_EMBED = 64
_BATCH = 16384
_HIST = 200

_GRP = 128
_GPC = 8
_CHUNK = _GRP * _GPC


def _make_gather(n_idx: int):
    info = plsc.get_sparse_core_info()
    nw = info.num_cores * info.num_subcores
    per_w = n_idx // nw
    assert per_w % _CHUNK == 0
    n_iter = per_w // _CHUNK
    groups_per_w = per_w // _GRP

    mesh = plsc.VectorSubcoreMesh(core_axis_name="c", subcore_axis_name="s")

    @functools.partial(
        pl.kernel,
        out_type=jax.ShapeDtypeStruct((n_idx, _EMBED), jnp.float32),
        mesh=mesh,
        scratch_types=[
            pltpu.VMEM((_GPC, _GRP), jnp.int32),
            pltpu.VMEM((_CHUNK, _EMBED), jnp.float32),
            pltpu.SemaphoreType.DMA,
        ],
        compiler_params=pltpu.CompilerParams(use_tc_tiling_on_sc=False),
    )
    def gather(table_hbm, idx_hbm, out_hbm, idx_v, rows_v, sem):
        wid = lax.axis_index("s") * info.num_cores + lax.axis_index("c")
        g_base = wid * groups_per_w
        r_base = wid * per_w

        def body(c, carry):
            pltpu.sync_copy(idx_hbm.at[pl.ds(g_base + c * _GPC, _GPC)], idx_v)
            copies = []
            for j in range(_GPC):
                copies.append(
                    pltpu.async_copy(
                        table_hbm.at[idx_v.at[j]],
                        rows_v.at[pl.ds(j * _GRP, _GRP)],
                        sem,
                    )
                )
            for cp in copies:
                cp.wait()
            pltpu.sync_copy(
                rows_v, out_hbm.at[pl.ds(r_base + c * _CHUNK, _CHUNK)]
            )
            return carry

        lax.fori_loop(0, n_iter, body, 0)

    return gather


def kernel(input_ids, table):
    n_idx = _BATCH * _HIST
    idx = input_ids.reshape(n_idx // _GRP, _GRP).astype(jnp.int32)
    out = _make_gather(n_idx)(table, idx)
    return out.reshape(_BATCH, _HIST, _EMBED)


# speedup vs baseline: 1.0282x; 1.0282x over previous
import functools

import jax
import jax.numpy as jnp
from jax import lax
from jax.experimental import pallas as pl
from jax.experimental.pallas import tpu as pltpu
from jax.experimental.pallas import tpu_sc as plsc

_VOCAB = 1000000
_EMBED = 64
_BATCH = 16384
_HIST = 200

_GRP = 128
_GPC = 5
_CHUNK = _GRP * _GPC
_NBUF = 2


def _make_gather(n_idx: int):
    info = plsc.get_sparse_core_info()
    nw = info.num_cores * info.num_subcores
    per_w = n_idx // nw
    assert per_w % (_CHUNK * _NBUF) == 0
    n_iter = per_w // _CHUNK

    mesh = plsc.VectorSubcoreMesh(core_axis_name="c", subcore_axis_name="s")

    @functools.partial(
        pl.kernel,
        out_type=jax.ShapeDtypeStruct((n_idx, _EMBED), jnp.float32),
        mesh=mesh,
        scratch_types=[
            pltpu.VMEM((_NBUF * _CHUNK,), jnp.int32),
            pltpu.VMEM((_NBUF * _CHUNK, _EMBED), jnp.float32),
            [pltpu.SemaphoreType.DMA] * _NBUF,
            [pltpu.SemaphoreType.DMA] * _NBUF,
            [pltpu.SemaphoreType.DMA] * _NBUF,
        ],
        compiler_params=pltpu.CompilerParams(use_tc_tiling_on_sc=False),
    )
    def gather(table_hbm, idx_hbm, out_hbm, idx_v, rows_v, gsem, ssem, isem):
        wid = lax.axis_index("s") * info.num_cores + lax.axis_index("c")
        r_base = wid * per_w

        def idx_copy(c, b, sem):
            i0 = r_base + jnp.minimum(c, n_iter - 1) * _CHUNK
            return pltpu.make_async_copy(
                idx_hbm.at[pl.ds(i0, _CHUNK)],
                idx_v.at[pl.ds(b * _CHUNK, _CHUNK)],
                sem,
            )

        def gather_copy(b, j, sem):
            return pltpu.make_async_copy(
                table_hbm.at[idx_v.at[pl.ds(b * _CHUNK + j * _GRP, _GRP)]],
                rows_v.at[pl.ds(b * _CHUNK + j * _GRP, _GRP)],
                sem,
            )

        def store_copy(c, b, sem):
            return pltpu.make_async_copy(
                rows_v.at[pl.ds(b * _CHUNK, _CHUNK)],
                out_hbm.at[pl.ds(r_base + c * _CHUNK, _CHUNK)],
                sem,
            )

        for b in range(_NBUF):
            idx_copy(b, b, isem[b]).start()

        def body(i, carry):
            for b in range(_NBUF):
                c = i * _NBUF + b
                idx_copy(c, b, isem[b]).wait()
                @pl.when(i > 0)
                def _():
                    store_copy(c - _NBUF, b, ssem[b]).wait()
                for j in range(_GPC):
                    gather_copy(b, j, gsem[b]).start()
                for j in range(_GPC):
                    gather_copy(b, j, gsem[b]).wait()
                idx_copy(c + _NBUF, b, isem[b]).start()
                store_copy(c, b, ssem[b]).start()
            return carry

        lax.fori_loop(0, n_iter // _NBUF, body, 0)

        for b in range(_NBUF):
            store_copy(n_iter - _NBUF + b, b, ssem[b]).wait()
            idx_copy(n_iter, b, isem[b]).wait()

    return gather


def kernel(input_ids, table):
    n_idx = _BATCH * _HIST
    idx = input_ids.reshape(n_idx).astype(jnp.int32)
    out = _make_gather(n_idx)(table, idx)
    return out.reshape(_BATCH, _HIST, _EMBED)


# speedup vs baseline: 1.0290x; 1.0008x over previous
import functools

import jax
import jax.numpy as jnp
from jax import lax
from jax.experimental import pallas as pl
from jax.experimental.pallas import tpu as pltpu
from jax.experimental.pallas import tpu_sc as plsc

_VOCAB = 1000000
_EMBED = 64
_BATCH = 16384
_HIST = 200

_GRP = 128
_GPC = 5
_CHUNK = _GRP * _GPC
_NBUF = 2


def _make_gather(n_idx: int):
    info = plsc.get_sparse_core_info()
    nw = info.num_cores * info.num_subcores
    per_w = n_idx // nw
    assert per_w % (_CHUNK * _NBUF) == 0
    n_iter = per_w // _CHUNK

    mesh = plsc.VectorSubcoreMesh(core_axis_name="c", subcore_axis_name="s")

    @functools.partial(
        pl.kernel,
        out_type=jax.ShapeDtypeStruct((n_idx, _EMBED), jnp.float32),
        mesh=mesh,
        scratch_types=[
            pltpu.VMEM((_NBUF * _CHUNK,), jnp.int32),
            pltpu.VMEM((_NBUF * _CHUNK, _EMBED), jnp.float32),
            [pltpu.SemaphoreType.DMA] * _NBUF,
            [pltpu.SemaphoreType.DMA] * _NBUF,
            [pltpu.SemaphoreType.DMA] * _NBUF,
        ],
        compiler_params=pltpu.CompilerParams(use_tc_tiling_on_sc=False),
    )
    def gather(table_hbm, idx_hbm, out_hbm, idx_v, rows_v, gsem, ssem, isem):
        wid = lax.axis_index("s") * info.num_cores + lax.axis_index("c")
        r_base = wid * per_w

        def idx_copy(c, b, sem):
            i0 = r_base + jnp.minimum(c, n_iter - 1) * _CHUNK
            return pltpu.make_async_copy(
                idx_hbm.at[pl.ds(i0, _CHUNK)],
                idx_v.at[pl.ds(b * _CHUNK, _CHUNK)],
                sem,
            )

        def gather_copy(b, j, sem):
            return pltpu.make_async_copy(
                table_hbm.at[idx_v.at[pl.ds(b * _CHUNK + j * _GRP, _GRP)]],
                rows_v.at[pl.ds(b * _CHUNK + j * _GRP, _GRP)],
                sem,
            )

        def gather_drain(b, sem):
            return pltpu.make_async_copy(
                table_hbm.at[pl.ds(0, _CHUNK)],
                rows_v.at[pl.ds(b * _CHUNK, _CHUNK)],
                sem,
            )

        def store_copy(c, b, sem):
            return pltpu.make_async_copy(
                rows_v.at[pl.ds(b * _CHUNK, _CHUNK)],
                out_hbm.at[pl.ds(r_base + c * _CHUNK, _CHUNK)],
                sem,
            )

        idx_copy(0, 0, isem[0]).start()

        def body(i, carry):
            for b in range(_NBUF):
                c = i * _NBUF + b
                bp = 1 - b
                @pl.when(i > 0)
                def _():
                    store_copy(c - _NBUF, b, ssem[b]).wait()
                idx_copy(c, b, isem[b]).wait()
                for j in range(_GPC):
                    gather_copy(b, j, gsem[b]).start()
                if b == 0:
                    @pl.when(i > 0)
                    def _():
                        gather_drain(bp, gsem[bp]).wait()
                        idx_copy(c + 1, bp, isem[bp]).start()
                        store_copy(c - 1, bp, ssem[bp]).start()
                    @pl.when(i == 0)
                    def _():
                        idx_copy(c + 1, bp, isem[bp]).start()
                else:
                    gather_drain(bp, gsem[bp]).wait()
                    idx_copy(c + 1, bp, isem[bp]).start()
                    store_copy(c - 1, bp, ssem[bp]).start()
            return carry

        lax.fori_loop(0, n_iter // _NBUF, body, 0)

        last = n_iter - 1
        bl = last % _NBUF
        gather_drain(bl, gsem[bl]).wait()
        store_copy(last, bl, ssem[bl]).start()
        store_copy(last - 1, 1 - bl, ssem[1 - bl]).wait()
        store_copy(last, bl, ssem[bl]).wait()
        idx_copy(n_iter, 1 - bl, isem[1 - bl]).wait()

    return gather


def kernel(input_ids, table):
    n_idx = _BATCH * _HIST
    idx = input_ids.reshape(n_idx).astype(jnp.int32)
    out = _make_gather(n_idx)(table, idx)
    return out.reshape(_BATCH, _HIST, _EMBED)


# speedup vs baseline: 1.6977x; 1.6498x over previous
import functools

import jax
import jax.numpy as jnp
from jax import lax
from jax.experimental import pallas as pl
from jax.experimental.pallas import tpu as pltpu
from jax.experimental.pallas import tpu_sc as plsc

_VOCAB = 1000000
_EMBED = 64
_BATCH = 16384
_HIST = 200

_GRP = 128
_GPC = 5
_CHUNK = _GRP * _GPC
_NBUF = 2


def _make_gather(n_idx: int):
    info = plsc.get_sparse_core_info()
    nw = info.num_cores * info.num_subcores
    per_w = n_idx // nw
    assert per_w % (_CHUNK * _NBUF) == 0
    n_iter = per_w // _CHUNK

    mesh = plsc.VectorSubcoreMesh(core_axis_name="c", subcore_axis_name="s")

    @functools.partial(
        pl.kernel,
        out_type=jax.ShapeDtypeStruct((n_idx, 2 * _EMBED), jnp.float32),
        mesh=mesh,
        scratch_types=[
            pltpu.VMEM((_NBUF * _CHUNK,), jnp.int32),
            pltpu.VMEM((_NBUF * _CHUNK, _EMBED), jnp.float32),
            [pltpu.SemaphoreType.DMA] * _NBUF,
            [pltpu.SemaphoreType.DMA] * _NBUF,
            [pltpu.SemaphoreType.DMA] * _NBUF,
        ],
        compiler_params=pltpu.CompilerParams(use_tc_tiling_on_sc=False),
    )
    def gather(table_hbm, idx_hbm, out_hbm, idx_v, rows_v, gsem, ssem, isem):
        wid = lax.axis_index("s") * info.num_cores + lax.axis_index("c")
        r_base = wid * per_w

        def idx_copy(c, b, sem):
            i0 = r_base + jnp.minimum(c, n_iter - 1) * _CHUNK
            return pltpu.make_async_copy(
                idx_hbm.at[pl.ds(i0, _CHUNK)],
                idx_v.at[pl.ds(b * _CHUNK, _CHUNK)],
                sem,
            )

        def gather_copy(b, j, sem):
            return pltpu.make_async_copy(
                table_hbm.at[idx_v.at[pl.ds(b * _CHUNK + j * _GRP, _GRP)]],
                rows_v.at[pl.ds(b * _CHUNK + j * _GRP, _GRP)],
                sem,
            )

        def gather_drain(b, sem):
            return pltpu.make_async_copy(
                table_hbm.at[pl.ds(0, _CHUNK)],
                rows_v.at[pl.ds(b * _CHUNK, _CHUNK)],
                sem,
            )

        def store_copy(c, b, sem):
            return pltpu.make_async_copy(
                rows_v.at[pl.ds(b * _CHUNK, _CHUNK)],
                out_hbm.at[pl.ds(r_base + c * _CHUNK, _CHUNK), pl.ds(0, _EMBED)],
                sem,
            )

        idx_copy(0, 0, isem[0]).start()

        def body(i, carry):
            for b in range(_NBUF):
                c = i * _NBUF + b
                bp = 1 - b
                @pl.when(i > 0)
                def _():
                    store_copy(c - _NBUF, b, ssem[b]).wait()
                idx_copy(c, b, isem[b]).wait()
                for j in range(_GPC):
                    gather_copy(b, j, gsem[b]).start()
                if b == 0:
                    @pl.when(i > 0)
                    def _():
                        gather_drain(bp, gsem[bp]).wait()
                        idx_copy(c + 1, bp, isem[bp]).start()
                        store_copy(c - 1, bp, ssem[bp]).start()
                    @pl.when(i == 0)
                    def _():
                        idx_copy(c + 1, bp, isem[bp]).start()
                else:
                    gather_drain(bp, gsem[bp]).wait()
                    idx_copy(c + 1, bp, isem[bp]).start()
                    store_copy(c - 1, bp, ssem[bp]).start()
            return carry

        lax.fori_loop(0, n_iter // _NBUF, body, 0)

        last = n_iter - 1
        bl = last % _NBUF
        gather_drain(bl, gsem[bl]).wait()
        store_copy(last, bl, ssem[bl]).start()
        store_copy(last - 1, 1 - bl, ssem[1 - bl]).wait()
        store_copy(last, bl, ssem[bl]).wait()
        idx_copy(n_iter, 1 - bl, isem[1 - bl]).wait()

    return gather


def kernel(input_ids, table):
    n_idx = _BATCH * _HIST
    idx = input_ids.reshape(n_idx).astype(jnp.int32)
    out = _make_gather(n_idx)(table, idx)
    return out[:, :_EMBED].reshape(_BATCH, _HIST, _EMBED)
